# baseline (speedup 1.0000x reference)
import functools

import jax
import jax.numpy as jnp
from jax import lax
from jax.experimental import pallas as pl
from jax.experimental.pallas import tpu as pltpu
from jax.experimental.pallas import tpu_sc as plsc

B = 16384
D_NUM = 13
N_CAT = 26
CARD = 10000
D_TOKEN = 64
NROW = 1 + D_NUM + N_CAT

NC = 2
NS = 16
NW = NC * NS
RPW = B // NW
CHUNK = 16
NCHUNK = RPW // CHUNK
FLAT = CHUNK * N_CAT
GSZ = 104
NG = FLAT // GSZ
LANES = 16
DSUB = D_TOKEN // LANES
OUTW = NROW * D_TOKEN


def _body(emb_hbm, xc_hbm, xn_hbm, w_hbm, bias_hbm, offs_hbm, out_hbm,
          w_v, bias_v, offs_v, xc_v, xn_v, idx_v, g_v, out_v,
          sem_in0, sem_in1, sem_g0, sem_g1, sem_out):
    cid = lax.axis_index("c")
    sid = lax.axis_index("s")
    wid = sid * NC + cid
    base = wid * RPW
    sem_in = (sem_in0, sem_in1)
    sem_g = (sem_g0, sem_g1)

    pltpu.sync_copy(w_hbm, w_v)
    pltpu.sync_copy(bias_hbm, bias_v)
    pltpu.sync_copy(offs_hbm, offs_v)

    def in_copies(ci, s):
        b0 = base + ci * CHUNK
        return (
            pltpu.make_async_copy(
                xc_hbm.at[pl.ds(b0 * N_CAT, FLAT)], xc_v.at[s], sem_in[s]),
            pltpu.make_async_copy(
                xn_hbm.at[pl.ds(b0 * LANES, CHUNK * LANES)], xn_v.at[s],
                sem_in[s]),
        )

    def gather_copies(s):
        return [
            pltpu.make_async_copy(
                emb_hbm.at[idx_v.at[s, pl.ds(g * GSZ, GSZ)]],
                g_v.at[s, pl.ds(g * GSZ, GSZ)],
                sem_g[s])
            for g in range(NG)
        ]

    def out_copy(ci):
        b0 = base + ci * CHUNK
        return pltpu.make_async_copy(
            out_v, out_hbm.at[pl.ds(b0, CHUNK)], sem_out)

    def compute_idx(s):
        for k in range(FLAT // LANES):
            sl = pl.ds(k * LANES, LANES)
            v = xc_v[s, sl]
            idx_v[s, sl] = jnp.where(v == -1, CARD - 1, v) + offs_v[sl]

    for cp in in_copies(0, 0):
        cp.start()
        cp.wait()
    compute_idx(0)
    for cp in gather_copies(0):
        cp.start()
    for cp in in_copies(1, 1):
        cp.start()

    def pair_body(k2, carry):
        for b in (0, 1):
            s = b
            s1 = 1 - b
            chunk = k2 * 2 + b

            @pl.when(chunk > 0)
            def _():
                out_copy(chunk - 1).wait()

            for jg in ((0, 1, 2, 3, 4), (5, 6, 7, 8), (9, 10, 11, 12)):
                wvs = {}
                for j in jg:
                    for d in range(DSUB):
                        w_sl = pl.ds((j + 1) * D_TOKEN + d * LANES, LANES)
                        wvs[(j, d)] = (w_v[w_sl], bias_v[w_sl])
                if jg[0] == 0:
                    w0 = [w_v[pl.ds(d * LANES, LANES)] for d in range(DSUB)]

                @plsc.parallel_loop(0, CHUNK, 1, unroll=4)
                def _(i):
                    xn_row = xn_v[s, pl.ds(i * LANES, LANES)]
                    if jg[0] == 0:
                        for d in range(DSUB):
                            out_v[i, pl.ds(d * LANES, LANES)] = w0[d]
                    for j in jg:
                        sc = xn_row[j]
                        for d in range(DSUB):
                            out_v[i, pl.ds((j + 1) * D_TOKEN + d * LANES,
                                           LANES)] = \
                                wvs[(j, d)][0] * sc + wvs[(j, d)][1]

            @pl.when(chunk + 2 < NCHUNK)
            def _():
                for cp in in_copies(chunk + 2, s):
                    cp.start()

            @pl.when(chunk + 1 < NCHUNK)
            def _():
                for cp in in_copies(chunk + 1, s1):
                    cp.wait()
                compute_idx(s1)
                for cp in gather_copies(s1):
                    cp.start()

            for cp in gather_copies(s):
                cp.wait()

            for c in range(N_CAT):
                r = 1 + D_NUM + c
                bvs = [bias_v[pl.ds(r * D_TOKEN + d * LANES, LANES)]
                       for d in range(DSUB)]

                @plsc.parallel_loop(0, CHUNK, 1, unroll=4)
                def _(i):
                    for d in range(DSUB):
                        ds = pl.ds(d * LANES, LANES)
                        out_v[i, pl.ds(r * D_TOKEN + d * LANES, LANES)] = \
                            g_v[s, i * N_CAT + c, ds] + bvs[d]

            out_copy(chunk).start()
        return carry

    lax.fori_loop(0, NCHUNK // 2, pair_body, 0)
    out_copy(NCHUNK - 1).wait()


BT = 512


def _unflatten_body(f_ref, o_ref):
    for r in range(NROW):
        o_ref[:, r, :] = f_ref[:, pl.ds(r * D_TOKEN, D_TOKEN)]


def _unflatten(out2d):
    return pl.pallas_call(
        _unflatten_body,
        out_shape=jax.ShapeDtypeStruct((B, NROW, D_TOKEN), jnp.float32),
        grid=(B // BT,),
        in_specs=[pl.BlockSpec((BT, OUTW), lambda i: (i, 0))],
        out_specs=pl.BlockSpec((BT, NROW, D_TOKEN), lambda i: (i, 0, 0)),
    )(out2d)


@jax.jit
def _tokenize(emb_table, xc_flat, xn_flat, w_flat, bias_flat, offs_rep):
    mesh = plsc.VectorSubcoreMesh(core_axis_name="c", subcore_axis_name="s")
    out2d = pl.kernel(
        _body,
        out_type=jax.ShapeDtypeStruct((B, OUTW), jnp.float32),
        mesh=mesh,
        compiler_params=pltpu.CompilerParams(use_tc_tiling_on_sc=False),
        scratch_types=[
            pltpu.VMEM(((D_NUM + 1) * D_TOKEN,), jnp.float32),
            pltpu.VMEM((NROW * D_TOKEN,), jnp.float32),
            pltpu.VMEM((FLAT,), jnp.int32),
            pltpu.VMEM((2, FLAT), jnp.int32),
            pltpu.VMEM((2, CHUNK * LANES), jnp.float32),
            pltpu.VMEM((2, FLAT), jnp.int32),
            pltpu.VMEM((2, FLAT, D_TOKEN), jnp.float32),
            pltpu.VMEM((CHUNK, OUTW), jnp.float32),
            pltpu.SemaphoreType.DMA,
            pltpu.SemaphoreType.DMA,
            pltpu.SemaphoreType.DMA,
            pltpu.SemaphoreType.DMA,
            pltpu.SemaphoreType.DMA,
        ],
    )(emb_table, xc_flat, xn_flat, w_flat, bias_flat, offs_rep)
    return _unflatten(out2d)


def kernel(x_num, x_cat, weight, bias_p, emb_table):
    xc_flat = x_cat.reshape(-1)
    xn_flat = jnp.pad(x_num, ((0, 0), (0, LANES - D_NUM))).reshape(-1)
    w_flat = weight.reshape(-1)
    bias_flat = jnp.concatenate(
        [jnp.zeros((1, D_TOKEN), dtype=bias_p.dtype), bias_p],
        axis=0).reshape(-1)
    offs_rep = jnp.tile(jnp.arange(N_CAT, dtype=jnp.int32) * CARD, CHUNK)
    return _tokenize(emb_table, xc_flat, xn_flat, w_flat, bias_flat, offs_rep)


# speedup vs baseline: 1.6477x; 1.6477x over previous
import functools

import jax
import jax.numpy as jnp
from jax import lax
from jax.experimental import pallas as pl
from jax.experimental.pallas import tpu as pltpu
from jax.experimental.pallas import tpu_sc as plsc

B = 16384
D_NUM = 13
N_CAT = 26
CARD = 10000
D_TOKEN = 64
NROW = 1 + D_NUM + N_CAT

NC = 2
NS = 16
NW = NC * NS
RPW = B // NW
VOCAB = N_CAT * CARD
CHUNK = 8
NCHUNK = RPW // CHUNK
FLAT = CHUNK * N_CAT
GSZ = 104
NG = FLAT // GSZ
PAIRW = 2 * D_TOKEN
LANES = 16
DSUB = D_TOKEN // LANES
OUTW = NROW * D_TOKEN


def _body(emb_hbm, xc_hbm, xn_hbm, w_hbm, bias_hbm, offs_hbm, out_hbm,
          w_v, bias_v, offs_v, xc_v0, xc_v1, xn_v0, xn_v1,
          idx_v0, idx_v1, par_v0, par_v1, g_v0, g_v1, out_v,
          sem_in0, sem_in1, sem_g0, sem_g1, sem_out):
    cid = lax.axis_index("c")
    sid = lax.axis_index("s")
    wid = sid * NC + cid
    base = wid * RPW
    sem_in = (sem_in0, sem_in1)
    sem_g = (sem_g0, sem_g1)
    xc_vs = (xc_v0, xc_v1)
    xn_vs = (xn_v0, xn_v1)
    idx_vs = (idx_v0, idx_v1)
    par_vs = (par_v0, par_v1)
    g_vs = (g_v0, g_v1)

    pltpu.sync_copy(w_hbm, w_v)
    pltpu.sync_copy(bias_hbm, bias_v)
    pltpu.sync_copy(offs_hbm, offs_v)

    def in_copies(ci, s):
        b0 = base + ci * CHUNK
        return (
            pltpu.make_async_copy(
                xc_hbm.at[pl.ds(b0 * N_CAT, FLAT)], xc_vs[s], sem_in[s]),
            pltpu.make_async_copy(
                xn_hbm.at[pl.ds(b0 * LANES, CHUNK * LANES)], xn_vs[s],
                sem_in[s]),
        )

    def gather_copies(s):
        return [
            pltpu.make_async_copy(
                emb_hbm.at[idx_vs[s].at[pl.ds(g * GSZ, GSZ)]],
                g_vs[s].at[pl.ds(g * GSZ, GSZ)],
                sem_g[s])
            for g in range(NG)
        ]

    def out_copy(ci):
        b0 = base + ci * CHUNK
        return pltpu.make_async_copy(
            out_v, out_hbm.at[pl.ds(b0, CHUNK)], sem_out)

    def compute_idx(s):
        for k in range(FLAT // LANES):
            sl = pl.ds(k * LANES, LANES)
            v = xc_vs[s][sl]
            full = jnp.where(v == -1, CARD - 1, v) + offs_v[sl]
            idx_vs[s][sl] = full >> 1
            par_vs[s][sl] = (full & 1) * D_TOKEN

    for cp in in_copies(0, 0):
        cp.start()
        cp.wait()
    compute_idx(0)
    for cp in gather_copies(0):
        cp.start()
    for cp in in_copies(1, 1):
        cp.start()

    def pair_body(k2, carry):
        for b in (0, 1):
            s = b
            s1 = 1 - b
            chunk = k2 * 2 + b

            @pl.when(chunk > 0)
            def _():
                out_copy(chunk - 1).wait()

            for jg in ((0, 1, 2, 3, 4), (5, 6, 7, 8), (9, 10, 11, 12)):
                wvs = {}
                for j in jg:
                    for d in range(DSUB):
                        w_sl = pl.ds((j + 1) * D_TOKEN + d * LANES, LANES)
                        wvs[(j, d)] = (w_v[w_sl], bias_v[w_sl])
                if jg[0] == 0:
                    w0 = [w_v[pl.ds(d * LANES, LANES)] for d in range(DSUB)]

                @plsc.parallel_loop(0, CHUNK, 1, unroll=4)
                def _(i):
                    xn_row = xn_vs[s][pl.ds(i * LANES, LANES)]
                    if jg[0] == 0:
                        for d in range(DSUB):
                            out_v[i, 0, pl.ds(d * LANES, LANES)] = w0[d]
                    for j in jg:
                        sc = xn_row[j]
                        for d in range(DSUB):
                            out_v[i, j + 1, pl.ds(d * LANES, LANES)] = \
                                wvs[(j, d)][0] * sc + wvs[(j, d)][1]

            @pl.when(chunk + 2 < NCHUNK)
            def _():
                for cp in in_copies(chunk + 2, s):
                    cp.start()

            @pl.when(chunk + 1 < NCHUNK)
            def _():
                for cp in in_copies(chunk + 1, s1):
                    cp.wait()
                compute_idx(s1)
                for cp in gather_copies(s1):
                    cp.start()

            for cp in gather_copies(s):
                cp.wait()

            for c in range(N_CAT):
                r = 1 + D_NUM + c
                bvs = [bias_v[pl.ds(r * D_TOKEN + d * LANES, LANES)]
                       for d in range(DSUB)]

                @plsc.parallel_loop(0, CHUNK, 1, unroll=4)
                def _(i):
                    fl = i * N_CAT + c
                    half = par_vs[s][pl.ds(fl, LANES)][0]
                    for d in range(DSUB):
                        out_v[i, r, pl.ds(d * LANES, LANES)] = \
                            g_vs[s][fl, pl.ds(half + d * LANES, LANES)] + bvs[d]

            out_copy(chunk).start()
        return carry

    lax.fori_loop(0, NCHUNK // 2, pair_body, 0)
    out_copy(NCHUNK - 1).wait()


@jax.jit
def _tokenize(emb_table, xc_flat, xn_flat, w_flat, bias_flat, offs_rep):
    mesh = plsc.VectorSubcoreMesh(core_axis_name="c", subcore_axis_name="s")
    out3d = pl.kernel(
        _body,
        out_type=jax.ShapeDtypeStruct((B, NROW, D_TOKEN), jnp.float32),
        mesh=mesh,
        scratch_types=[
            pltpu.VMEM(((D_NUM + 1) * D_TOKEN,), jnp.float32),
            pltpu.VMEM((NROW * D_TOKEN,), jnp.float32),
            pltpu.VMEM((FLAT,), jnp.int32),
            pltpu.VMEM((FLAT,), jnp.int32),
            pltpu.VMEM((FLAT,), jnp.int32),
            pltpu.VMEM((CHUNK * LANES,), jnp.float32),
            pltpu.VMEM((CHUNK * LANES,), jnp.float32),
            pltpu.VMEM((FLAT,), jnp.int32),
            pltpu.VMEM((FLAT,), jnp.int32),
            pltpu.VMEM((FLAT + LANES,), jnp.int32),
            pltpu.VMEM((FLAT + LANES,), jnp.int32),
            pltpu.VMEM((FLAT, PAIRW), jnp.float32),
            pltpu.VMEM((FLAT, PAIRW), jnp.float32),
            pltpu.VMEM((CHUNK, NROW, D_TOKEN), jnp.float32),
            pltpu.SemaphoreType.DMA,
            pltpu.SemaphoreType.DMA,
            pltpu.SemaphoreType.DMA,
            pltpu.SemaphoreType.DMA,
            pltpu.SemaphoreType.DMA,
        ],
    )(emb_table, xc_flat, xn_flat, w_flat, bias_flat, offs_rep)
    return out3d


def kernel(x_num, x_cat, weight, bias_p, emb_table):
    emb_pairs = emb_table.reshape(VOCAB // 2, PAIRW)
    xc_flat = x_cat.reshape(-1)
    xn_flat = jnp.pad(x_num, ((0, 0), (0, LANES - D_NUM))).reshape(-1)
    w_flat = weight.reshape(-1)
    bias_flat = jnp.concatenate(
        [jnp.zeros((1, D_TOKEN), dtype=bias_p.dtype), bias_p],
        axis=0).reshape(-1)
    offs_rep = jnp.tile(jnp.arange(N_CAT, dtype=jnp.int32) * CARD, CHUNK)
    return _tokenize(emb_pairs, xc_flat, xn_flat, w_flat, bias_flat, offs_rep)


# speedup vs baseline: 1.9041x; 1.1556x over previous
import functools

import jax
import jax.numpy as jnp
from jax import lax
from jax.experimental import pallas as pl
from jax.experimental.pallas import tpu as pltpu
from jax.experimental.pallas import tpu_sc as plsc

B = 16384
D_NUM = 13
N_CAT = 26
CARD = 10000
D_TOKEN = 64
NROW = 1 + D_NUM + N_CAT

NC = 2
NS = 16
NW = NC * NS
RPW = B // NW
VOCAB = N_CAT * CARD
CHUNK = 8
NCHUNK = RPW // CHUNK
FLAT = CHUNK * N_CAT
GSZ = 104
NG = FLAT // GSZ
PAIRW = 2 * D_TOKEN
LANES = 16
DSUB = D_TOKEN // LANES
OUTW = NROW * D_TOKEN


def _body(emb_hbm, xc_hbm, xn_hbm, w_hbm, bias_hbm, offs_hbm, out_hbm,
          w_v, bias_v, offs_v, xc_v0, xc_v1, xn_v0, xn_v1,
          idx_v0, idx_v1, g_v0, g_v1, out_v,
          sem_in0, sem_in1, sem_g0, sem_g1, sem_out):
    cid = lax.axis_index("c")
    sid = lax.axis_index("s")
    wid = sid * NC + cid
    base = wid * RPW
    sem_in = (sem_in0, sem_in1)
    sem_g = (sem_g0, sem_g1)
    xc_vs = (xc_v0, xc_v1)
    xn_vs = (xn_v0, xn_v1)
    idx_vs = (idx_v0, idx_v1)
    g_vs = (g_v0, g_v1)

    pltpu.sync_copy(w_hbm, w_v)
    pltpu.sync_copy(bias_hbm, bias_v)
    pltpu.sync_copy(offs_hbm, offs_v)

    def in_copies(ci, s):
        b0 = base + ci * CHUNK
        return (
            pltpu.make_async_copy(
                xc_hbm.at[pl.ds(b0 * N_CAT, FLAT)], xc_vs[s], sem_in[s]),
            pltpu.make_async_copy(
                xn_hbm.at[pl.ds(b0 * LANES, CHUNK * LANES)], xn_vs[s],
                sem_in[s]),
        )

    def gather_copies(s):
        return [
            pltpu.make_async_copy(
                emb_hbm.at[idx_vs[s].at[pl.ds(g * GSZ, GSZ)]],
                g_vs[s].at[pl.ds(g * GSZ, GSZ)],
                sem_g[s])
            for g in range(NG)
        ]

    def out_copy(ci):
        b0 = base + ci * CHUNK
        return pltpu.make_async_copy(
            out_v, out_hbm.at[pl.ds(b0, CHUNK)], sem_out)

    def compute_idx(s):
        for k in range(FLAT // LANES):
            sl = pl.ds(k * LANES, LANES)
            v = xc_vs[s][sl]
            idx_vs[s][sl] = jnp.where(v == -1, CARD - 1, v) + offs_v[sl]

    for cp in in_copies(0, 0):
        cp.start()
        cp.wait()
    compute_idx(0)
    for cp in gather_copies(0):
        cp.start()
    for cp in in_copies(1, 1):
        cp.start()

    def pair_body(k2, carry):
        for b in (0, 1):
            s = b
            s1 = 1 - b
            chunk = k2 * 2 + b

            @pl.when(chunk > 0)
            def _():
                out_copy(chunk - 1).wait()

            for jg in ((0, 1, 2, 3, 4), (5, 6, 7, 8), (9, 10, 11, 12)):
                wvs = {}
                for j in jg:
                    for d in range(DSUB):
                        w_sl = pl.ds((j + 1) * D_TOKEN + d * LANES, LANES)
                        wvs[(j, d)] = (w_v[w_sl], bias_v[w_sl])
                if jg[0] == 0:
                    w0 = [w_v[pl.ds(d * LANES, LANES)] for d in range(DSUB)]

                @plsc.parallel_loop(0, CHUNK, 1, unroll=4)
                def _(i):
                    xn_row = xn_vs[s][pl.ds(i * LANES, LANES)]
                    if jg[0] == 0:
                        for d in range(DSUB):
                            out_v[i, 0, pl.ds(d * LANES, LANES)] = w0[d]
                    for j in jg:
                        sc = xn_row[j]
                        for d in range(DSUB):
                            out_v[i, j + 1, pl.ds(d * LANES, LANES)] = \
                                wvs[(j, d)][0] * sc + wvs[(j, d)][1]

            @pl.when(chunk + 2 < NCHUNK)
            def _():
                for cp in in_copies(chunk + 2, s):
                    cp.start()

            @pl.when(chunk + 1 < NCHUNK)
            def _():
                for cp in in_copies(chunk + 1, s1):
                    cp.wait()
                compute_idx(s1)
                for cp in gather_copies(s1):
                    cp.start()

            for cp in gather_copies(s):
                cp.wait()

            for c in range(N_CAT):
                r = 1 + D_NUM + c
                bvs = [bias_v[pl.ds(r * D_TOKEN + d * LANES, LANES)]
                       for d in range(DSUB)]

                @plsc.parallel_loop(0, CHUNK, 1, unroll=4)
                def _(i):
                    fl = i * N_CAT + c
                    for d in range(DSUB):
                        out_v[i, r, pl.ds(d * LANES, LANES)] = \
                            g_vs[s][fl, pl.ds(d * LANES, LANES)] + bvs[d]

            out_copy(chunk).start()
        return carry

    lax.fori_loop(0, NCHUNK // 2, pair_body, 0)
    out_copy(NCHUNK - 1).wait()


@jax.jit
def _tokenize(emb_table, xc_flat, xn_flat, w_flat, bias_flat, offs_rep):
    mesh = plsc.VectorSubcoreMesh(core_axis_name="c", subcore_axis_name="s")
    out3d = pl.kernel(
        _body,
        out_type=jax.ShapeDtypeStruct((B, NROW, D_TOKEN), jnp.float32),
        mesh=mesh,
        scratch_types=[
            pltpu.VMEM(((D_NUM + 1) * D_TOKEN,), jnp.float32),
            pltpu.VMEM((NROW * D_TOKEN,), jnp.float32),
            pltpu.VMEM((FLAT,), jnp.int32),
            pltpu.VMEM((FLAT,), jnp.int32),
            pltpu.VMEM((FLAT,), jnp.int32),
            pltpu.VMEM((CHUNK * LANES,), jnp.float32),
            pltpu.VMEM((CHUNK * LANES,), jnp.float32),
            pltpu.VMEM((FLAT,), jnp.int32),
            pltpu.VMEM((FLAT,), jnp.int32),
            pltpu.VMEM((FLAT, PAIRW), jnp.float32),
            pltpu.VMEM((FLAT, PAIRW), jnp.float32),
            pltpu.VMEM((CHUNK, NROW, D_TOKEN), jnp.float32),
            pltpu.SemaphoreType.DMA,
            pltpu.SemaphoreType.DMA,
            pltpu.SemaphoreType.DMA,
            pltpu.SemaphoreType.DMA,
            pltpu.SemaphoreType.DMA,
        ],
    )(emb_table, xc_flat, xn_flat, w_flat, bias_flat, offs_rep)
    return out3d


def kernel(x_num, x_cat, weight, bias_p, emb_table):
    emb_wide = jnp.pad(emb_table, ((0, 0), (0, PAIRW - D_TOKEN)))
    xc_flat = x_cat.reshape(-1)
    xn_flat = jnp.pad(x_num, ((0, 0), (0, LANES - D_NUM))).reshape(-1)
    w_flat = weight.reshape(-1)
    bias_flat = jnp.concatenate(
        [jnp.zeros((1, D_TOKEN), dtype=bias_p.dtype), bias_p],
        axis=0).reshape(-1)
    offs_rep = jnp.tile(jnp.arange(N_CAT, dtype=jnp.int32) * CARD, CHUNK)
    return _tokenize(emb_wide, xc_flat, xn_flat, w_flat, bias_flat, offs_rep)


# speedup vs baseline: 1.9056x; 1.0008x over previous
import jax
import jax.numpy as jnp
from jax import lax
from jax.experimental import pallas as pl
from jax.experimental.pallas import tpu as pltpu
from jax.experimental.pallas import tpu_sc as plsc

B = 16384
D_NUM = 13
N_CAT = 26
CARD = 10000
D_TOKEN = 64
NROW = 1 + D_NUM + N_CAT

NC = 2
NS = 16
NW = NC * NS
RPW = B // NW
VOCAB = N_CAT * CARD
CHUNK = 8
NCHUNK = RPW // CHUNK
FLAT = CHUNK * N_CAT
GSZ = 104
NG = FLAT // GSZ
PAIRW = 2 * D_TOKEN
LANES = 16
DSUB = D_TOKEN // LANES
OUTW = NROW * D_TOKEN


def _body(emb_hbm, xc_hbm, xn_hbm, w_hbm, bias_hbm, offs_hbm, out_hbm,
          w_v, bias_v, offs_v, xc_v0, xc_v1, xn_v0, xn_v1,
          idx_v0, idx_v1, g_v0, g_v1, out_v,
          sem_in0, sem_in1, sem_g0, sem_g1, sem_out):
    cid = lax.axis_index("c")
    sid = lax.axis_index("s")
    wid = sid * NC + cid
    base = wid * RPW
    sem_in = (sem_in0, sem_in1)
    sem_g = (sem_g0, sem_g1)
    xc_vs = (xc_v0, xc_v1)
    xn_vs = (xn_v0, xn_v1)
    idx_vs = (idx_v0, idx_v1)
    g_vs = (g_v0, g_v1)

    pltpu.sync_copy(w_hbm, w_v)
    pltpu.sync_copy(bias_hbm, bias_v)
    pltpu.sync_copy(offs_hbm, offs_v)

    def in_copies(ci, s):
        b0 = base + ci * CHUNK
        return (
            pltpu.make_async_copy(
                xc_hbm.at[pl.ds(b0 * N_CAT, FLAT)], xc_vs[s], sem_in[s]),
            pltpu.make_async_copy(
                xn_hbm.at[pl.ds(b0 * LANES, CHUNK * LANES)], xn_vs[s],
                sem_in[s]),
        )

    def gather_copies(s):
        return [
            pltpu.make_async_copy(
                emb_hbm.at[idx_vs[s].at[pl.ds(g * GSZ, GSZ)]],
                g_vs[s].at[pl.ds(g * GSZ, GSZ)],
                sem_g[s])
            for g in range(NG)
        ]

    def out_copy(ci):
        b0 = base + ci * CHUNK
        return pltpu.make_async_copy(
            out_v, out_hbm.at[pl.ds(b0, CHUNK)], sem_out)

    def compute_idx(s):
        for k in range(FLAT // LANES):
            sl = pl.ds(k * LANES, LANES)
            v = xc_vs[s][sl]
            idx_vs[s][sl] = jnp.where(v == -1, CARD - 1, v) + offs_v[sl]

    for cp in in_copies(0, 0):
        cp.start()
        cp.wait()
    compute_idx(0)
    for cp in gather_copies(0):
        cp.start()
    for cp in in_copies(1, 1):
        cp.start()

    def pair_body(k2, carry):
        for b in (0, 1):
            s = b
            s1 = 1 - b
            chunk = k2 * 2 + b

            @pl.when(chunk > 0)
            def _():
                out_copy(chunk - 1).wait()

            for jg in ((0, 1, 2, 3, 4), (5, 6, 7, 8), (9, 10, 11, 12)):
                wvs = {}
                for j in jg:
                    for d in range(DSUB):
                        w_sl = pl.ds((j + 1) * D_TOKEN + d * LANES, LANES)
                        wvs[(j, d)] = (w_v[w_sl], bias_v[w_sl])
                if jg[0] == 0:
                    w0 = [w_v[pl.ds(d * LANES, LANES)] for d in range(DSUB)]

                @plsc.parallel_loop(0, CHUNK, 1, unroll=4)
                def _(i):
                    xn_row = xn_vs[s][pl.ds(i * LANES, LANES)]
                    if jg[0] == 0:
                        for d in range(DSUB):
                            out_v[i, 0, pl.ds(d * LANES, LANES)] = w0[d]
                    for j in jg:
                        sc = xn_row[j]
                        for d in range(DSUB):
                            out_v[i, j + 1, pl.ds(d * LANES, LANES)] = \
                                wvs[(j, d)][0] * sc + wvs[(j, d)][1]

            @pl.when(chunk + 2 < NCHUNK)
            def _():
                for cp in in_copies(chunk + 2, s):
                    cp.start()

            @pl.when(chunk + 1 < NCHUNK)
            def _():
                for cp in in_copies(chunk + 1, s1):
                    cp.wait()
                compute_idx(s1)
                for cp in gather_copies(s1):
                    cp.start()

            for cp in gather_copies(s):
                cp.wait()

            for c in range(N_CAT):
                r = 1 + D_NUM + c
                bvs = [bias_v[pl.ds(r * D_TOKEN + d * LANES, LANES)]
                       for d in range(DSUB)]

                @plsc.parallel_loop(0, CHUNK, 1, unroll=4)
                def _(i):
                    fl = i * N_CAT + c
                    for d in range(DSUB):
                        out_v[i, r, pl.ds(d * LANES, LANES)] = \
                            g_vs[s][fl, pl.ds(d * LANES, LANES)] + bvs[d]

            out_copy(chunk).start()
        return carry

    lax.fori_loop(0, NCHUNK // 2, pair_body, 0)
    out_copy(NCHUNK - 1).wait()


@jax.jit
def _tokenize(emb_table, xc_flat, xn_flat, w_flat, bias_flat, offs_rep):
    mesh = plsc.VectorSubcoreMesh(core_axis_name="c", subcore_axis_name="s")
    out3d = pl.kernel(
        _body,
        out_type=jax.ShapeDtypeStruct((B, NROW, D_TOKEN), jnp.float32),
        mesh=mesh,
        scratch_types=[
            pltpu.VMEM(((D_NUM + 1) * D_TOKEN,), jnp.float32),
            pltpu.VMEM((NROW * D_TOKEN,), jnp.float32),
            pltpu.VMEM((FLAT,), jnp.int32),
            pltpu.VMEM((FLAT,), jnp.int32),
            pltpu.VMEM((FLAT,), jnp.int32),
            pltpu.VMEM((CHUNK * LANES,), jnp.float32),
            pltpu.VMEM((CHUNK * LANES,), jnp.float32),
            pltpu.VMEM((FLAT,), jnp.int32),
            pltpu.VMEM((FLAT,), jnp.int32),
            pltpu.VMEM((FLAT, PAIRW), jnp.float32),
            pltpu.VMEM((FLAT, PAIRW), jnp.float32),
            pltpu.VMEM((CHUNK, NROW, D_TOKEN), jnp.float32),
            pltpu.SemaphoreType.DMA,
            pltpu.SemaphoreType.DMA,
            pltpu.SemaphoreType.DMA,
            pltpu.SemaphoreType.DMA,
            pltpu.SemaphoreType.DMA,
        ],
    )(emb_table, xc_flat, xn_flat, w_flat, bias_flat, offs_rep)
    return out3d


def kernel(x_num, x_cat, weight, bias_p, emb_table):
    emb_wide = jnp.pad(emb_table, ((0, 0), (0, PAIRW - D_TOKEN)))
    xc_flat = x_cat.reshape(-1)
    xn_flat = jnp.pad(x_num, ((0, 0), (0, LANES - D_NUM))).reshape(-1)
    w_flat = weight.reshape(-1)
    bias_flat = jnp.concatenate(
        [jnp.zeros((1, D_TOKEN), dtype=bias_p.dtype), bias_p],
        axis=0).reshape(-1)
    offs_rep = jnp.tile(jnp.arange(N_CAT, dtype=jnp.int32) * CARD, CHUNK)
    return _tokenize(emb_wide, xc_flat, xn_flat, w_flat, bias_flat, offs_rep)
